# baseline (speedup 1.0000x reference)
import functools

import jax
import jax.numpy as jnp
from jax import lax
from jax.experimental import pallas as pl
from jax.experimental.pallas import tpu as pltpu
from jax.experimental.pallas import tpu_sc as plsc

FIELD = 26
BATCH = 16384
LANES = 16
NC = 2
NS = 16
NW = NC * NS
ROWS_W = BATCH // NW
GROUPS = ROWS_W // LANES
FLAT = FIELD * ROWS_W


def _sc_body(ids_hbm, vals_hbm, w_hbm, b_hbm, out_hbm,
             idx_v, vals_v, g_v, acc_v, out_v, b_v, sems):
    c = lax.axis_index("c")
    s = lax.axis_index("s")
    wid = s * NC + c
    base = wid * ROWS_W

    pltpu.sync_copy(ids_hbm.at[wid], idx_v)
    for j in range(FIELD):
        pltpu.async_copy(
            w_hbm.at[idx_v.at[pl.ds(j * ROWS_W, ROWS_W)]],
            g_v.at[pl.ds(j * ROWS_W, ROWS_W)],
            sems.at[j])
    pltpu.sync_copy(vals_hbm.at[pl.ds(base, ROWS_W), :], vals_v)
    pltpu.sync_copy(b_hbm, b_v)

    zero = jnp.zeros((LANES,), jnp.float32)
    for t in range(GROUPS):
        acc_v[pl.ds(t * LANES, LANES)] = zero

    lane_iota = lax.iota(jnp.int32, LANES)

    def field(j, col):
        off = j * ROWS_W
        pltpu.make_async_copy(
            w_hbm.at[idx_v.at[pl.ds(off, ROWS_W)]],
            g_v.at[pl.ds(off, ROWS_W)],
            sems.at[j]).wait()
        for t in range(GROUPS):
            w = g_v[pl.ds(off + t * LANES, LANES)]
            v = plsc.load_gather(vals_v, [lane_iota + (t * LANES), col])
            plsc.addupdate(acc_v.at[pl.ds(t * LANES, LANES)], w * v)
        return col + 1

    lax.fori_loop(0, FIELD, field, jnp.zeros((LANES,), jnp.int32))

    bias = b_v[...]
    for t in range(GROUPS):
        z = acc_v[pl.ds(t * LANES, LANES)] + bias
        out_v[pl.ds(t * LANES, LANES)] = 1.0 / (1.0 + jnp.exp(-z))
    pltpu.sync_copy(out_v, out_hbm.at[pl.ds(base, ROWS_W)])


_sc_kernel = functools.partial(
    pl.kernel,
    out_type=jax.ShapeDtypeStruct((BATCH,), jnp.float32),
    mesh=plsc.VectorSubcoreMesh(core_axis_name="c", subcore_axis_name="s"),
    compiler_params=pltpu.CompilerParams(
        needs_layout_passes=False, use_tc_tiling_on_sc=True),
    scratch_types=[
        pltpu.VMEM((FLAT,), jnp.int32),
        pltpu.VMEM((ROWS_W, FIELD), jnp.float32),
        pltpu.VMEM((FLAT,), jnp.float32),
        pltpu.VMEM((ROWS_W,), jnp.float32),
        pltpu.VMEM((ROWS_W,), jnp.float32),
        pltpu.VMEM((LANES,), jnp.float32),
        pltpu.SemaphoreType.DMA((FIELD,)),
    ],
)(_sc_body)


def kernel(feat_ids, feat_vals, LR_W, LR_B):
    ids_t = feat_ids.reshape(NW, ROWS_W, FIELD).transpose(0, 2, 1).reshape(NW, FLAT)
    b16 = jnp.broadcast_to(LR_B, (LANES,))
    return _sc_kernel(ids_t, feat_vals, LR_W, b16)


# speedup vs baseline: 1.1220x; 1.1220x over previous
import functools

import jax
import jax.numpy as jnp
from jax import lax
from jax.experimental import pallas as pl
from jax.experimental.pallas import tpu as pltpu
from jax.experimental.pallas import tpu_sc as plsc

FIELD = 26
BATCH = 16384
LANES = 16
NC = 2
NS = 16
NW = NC * NS
ROWS_W = BATCH // NW
GROUPS = ROWS_W // LANES
PACKED = 2 * FIELD * ROWS_W
HALF_FIELDS = FIELD // 2


def _sc_body(packed_hbm, w_hbm, b_hbm, out_hbm,
             buf_v, g_v, acc_v, out_v, b_v, sems):
    c = lax.axis_index("c")
    s = lax.axis_index("s")
    wid = s * NC + c

    half = HALF_FIELDS * 2 * ROWS_W
    pltpu.sync_copy(packed_hbm.at[wid, pl.ds(0, half)], buf_v.at[pl.ds(0, half)])
    for j in range(HALF_FIELDS):
        pltpu.async_copy(
            w_hbm.at[buf_v.at[pl.ds(2 * j * ROWS_W, ROWS_W)]],
            g_v.at[pl.ds(j * ROWS_W, ROWS_W)],
            sems.at[j])
    pltpu.sync_copy(packed_hbm.at[wid, pl.ds(half, PACKED - half)],
                    buf_v.at[pl.ds(half, PACKED - half)])
    for j in range(HALF_FIELDS, FIELD):
        pltpu.async_copy(
            w_hbm.at[buf_v.at[pl.ds(2 * j * ROWS_W, ROWS_W)]],
            g_v.at[pl.ds(j * ROWS_W, ROWS_W)],
            sems.at[j])
    pltpu.sync_copy(b_hbm, b_v)

    zero = jnp.zeros((LANES,), jnp.float32)
    for t in range(GROUPS):
        acc_v[pl.ds(t * LANES, LANES)] = zero

    for j in range(FIELD):
        off = j * ROWS_W
        voff = (2 * j + 1) * ROWS_W
        pltpu.make_async_copy(
            w_hbm.at[buf_v.at[pl.ds(2 * j * ROWS_W, ROWS_W)]],
            g_v.at[pl.ds(off, ROWS_W)],
            sems.at[j]).wait()

        def grp(t, carry, off=off, voff=voff):
            o = t * LANES
            w = g_v[pl.ds(off + o, LANES)]
            v = plsc.bitcast(buf_v[pl.ds(voff + o, LANES)], jnp.float32)
            plsc.addupdate(acc_v.at[pl.ds(o, LANES)], w * v)
            return carry

        lax.fori_loop(0, GROUPS, grp, 0)

    bias = b_v[...]

    def fin(t, carry):
        o = t * LANES
        z = acc_v[pl.ds(o, LANES)] + bias
        out_v[pl.ds(o, LANES)] = 1.0 / (1.0 + jnp.exp(-z))
        return carry

    lax.fori_loop(0, GROUPS, fin, 0)
    pltpu.sync_copy(out_v, out_hbm.at[pl.ds(wid * ROWS_W, ROWS_W)])


_sc_kernel = functools.partial(
    pl.kernel,
    out_type=jax.ShapeDtypeStruct((BATCH,), jnp.float32),
    mesh=plsc.VectorSubcoreMesh(core_axis_name="c", subcore_axis_name="s"),
    compiler_params=pltpu.CompilerParams(needs_layout_passes=False),
    scratch_types=[
        pltpu.VMEM((PACKED,), jnp.int32),
        pltpu.VMEM((FIELD * ROWS_W,), jnp.float32),
        pltpu.VMEM((ROWS_W,), jnp.float32),
        pltpu.VMEM((ROWS_W,), jnp.float32),
        pltpu.VMEM((LANES,), jnp.float32),
        pltpu.SemaphoreType.DMA((FIELD,)),
    ],
)(_sc_body)


def kernel(feat_ids, feat_vals, LR_W, LR_B):
    vals_bits = lax.bitcast_convert_type(feat_vals, jnp.int32)
    st = jnp.stack([feat_ids, vals_bits], axis=-1)
    packed = (st.reshape(NW, ROWS_W, 2 * FIELD)
                .transpose(0, 2, 1)
                .reshape(NW, PACKED))
    b16 = jnp.broadcast_to(LR_B, (LANES,))
    return _sc_kernel(packed, LR_W, b16)


# speedup vs baseline: 1.3664x; 1.2178x over previous
import functools

import jax
import jax.numpy as jnp
from jax import lax
from jax.experimental import pallas as pl
from jax.experimental.pallas import tpu as pltpu
from jax.experimental.pallas import tpu_sc as plsc

FIELD = 26
BATCH = 16384
LANES = 16
NC = 2
NS = 16
NW = NC * NS
ROWS_W = BATCH // NW
GROUPS = ROWS_W // LANES
FLAT = FIELD * ROWS_W


def _sc_body(ids_hbm, vals_hbm, w_hbm, b_hbm, out_hbm,
             idx_v, vals_v, g_v, acc_v, out_v, b_v, sems):
    c = lax.axis_index("c")
    s = lax.axis_index("s")
    wid = s * NC + c

    pltpu.sync_copy(ids_hbm.at[wid], idx_v)
    pltpu.sync_copy(vals_hbm.at[wid], vals_v)
    pltpu.sync_copy(b_hbm, b_v.at[pl.ds(0, 1)])
    for j in range(FIELD):
        pltpu.async_copy(
            w_hbm.at[idx_v.at[pl.ds(j * ROWS_W, ROWS_W)]],
            g_v.at[pl.ds(j * ROWS_W, ROWS_W)],
            sems.at[j])

    zero = jnp.zeros((LANES,), jnp.float32)
    for t in range(GROUPS):
        acc_v[pl.ds(t * LANES, LANES)] = zero

    def field(j, carry):
        off = j * ROWS_W
        pltpu.make_async_copy(
            w_hbm.at[idx_v.at[pl.ds(off, ROWS_W)]],
            g_v.at[pl.ds(off, ROWS_W)],
            sems.at[j]).wait()
        for t in range(GROUPS):
            w = g_v[pl.ds(off + t * LANES, LANES)]
            v = vals_v[pl.ds(off + t * LANES, LANES)]
            plsc.addupdate(acc_v.at[pl.ds(t * LANES, LANES)], w * v)
        return carry

    lax.fori_loop(0, FIELD, field, 0)

    bias = plsc.load_gather(b_v, [jnp.zeros((LANES,), jnp.int32)])
    for t in range(GROUPS):
        z = acc_v[pl.ds(t * LANES, LANES)] + bias
        out_v[pl.ds(t * LANES, LANES)] = 1.0 / (1.0 + jnp.exp(-z))
    pltpu.sync_copy(out_v, out_hbm.at[pl.ds(wid * ROWS_W, ROWS_W)])


_sc_kernel = functools.partial(
    pl.kernel,
    out_type=jax.ShapeDtypeStruct((BATCH,), jnp.float32),
    mesh=plsc.VectorSubcoreMesh(core_axis_name="c", subcore_axis_name="s"),
    compiler_params=pltpu.CompilerParams(needs_layout_passes=False),
    scratch_types=[
        pltpu.VMEM((FLAT,), jnp.int32),
        pltpu.VMEM((FLAT,), jnp.float32),
        pltpu.VMEM((FLAT,), jnp.float32),
        pltpu.VMEM((ROWS_W,), jnp.float32),
        pltpu.VMEM((ROWS_W,), jnp.float32),
        pltpu.VMEM((LANES,), jnp.float32),
        pltpu.SemaphoreType.DMA((FIELD,)),
    ],
)(_sc_body)


def kernel(feat_ids, feat_vals, LR_W, LR_B):
    ids_t = feat_ids.reshape(NW, ROWS_W, FIELD).transpose(0, 2, 1).reshape(NW, FLAT)
    vals_t = feat_vals.reshape(NW, ROWS_W, FIELD).transpose(0, 2, 1).reshape(NW, FLAT)
    return _sc_kernel(ids_t, vals_t, LR_W, LR_B)
